# baseline (speedup 1.0000x reference)
import jax
import jax.numpy as jnp
from jax import lax
from jax.experimental import pallas as pl
from jax.experimental.pallas import tpu as pltpu
from jax.experimental.pallas import tpu_sc as plsc

B = 16384
V = 1000000
R = 1000
D = 64
NC = 2
NS = 16
NW = NC * NS
BW = B // NW
IC = 128
NCH = BW // IC
NBUF = 8


def _triples(j, b, ent_hbm, rel_hbm, hb_v, tb_v, rb_v, hd_v, td_v, rd_v,
             sems):
    out = []
    for c in range(NCH):
        sl = pl.ds(c * IC, IC)
        out.append((ent_hbm.at[pl.ds(j * V, V)].at[hb_v.at[c]],
                    hd_v.at[b, sl], sems.at[b]))
        out.append((ent_hbm.at[pl.ds(j * V, V)].at[tb_v.at[c]],
                    td_v.at[b, sl], sems.at[b]))
        out.append((rel_hbm.at[pl.ds(j * R, R)].at[rb_v.at[c]],
                    rd_v.at[b, sl], sems.at[b]))
    return out


def _body(h_idx_hbm, t_idx_hbm, r_idx_hbm, ent_hbm, rel_hbm, out_hbm,
          hb_v, tb_v, rb_v, hd_v, td_v, rd_v, acc_v, out_v, sems):
    cid = lax.axis_index("c")
    sid = lax.axis_index("s")
    wid = sid * NC + cid
    row0 = wid * NCH

    pltpu.sync_copy(h_idx_hbm.at[pl.ds(row0, NCH)], hb_v)
    pltpu.sync_copy(t_idx_hbm.at[pl.ds(row0, NCH)], tb_v)
    pltpu.sync_copy(r_idx_hbm.at[pl.ds(row0, NCH)], rb_v)

    zero = jnp.zeros((16,), jnp.float32)
    for ch in range(BW // 16):
        acc_v[pl.ds(ch * 16, 16)] = zero

    def fire(j, b):
        for src, dst, sem in _triples(j, b, ent_hbm, rel_hbm, hb_v, tb_v,
                                      rb_v, hd_v, td_v, rd_v, sems):
            pltpu.async_copy(src, dst, sem)

    def wait_and_accum(j, b):
        for src, dst, sem in _triples(j, b, ent_hbm, rel_hbm, hb_v, tb_v,
                                      rb_v, hd_v, td_v, rd_v, sems):
            pltpu.make_async_copy(src, dst, sem).wait()
        for ch in range(BW // 16):
            sl = pl.ds(ch * 16, 16)
            acc_v[sl] = acc_v[sl] + jnp.abs(
                hd_v[b, sl] + rd_v[b, sl] - td_v[b, sl])

    for b in range(NBUF):
        fire(b, b)

    def grp_body(g, _):
        for b in range(NBUF):
            j = g * NBUF + b
            wait_and_accum(j, b)
            fire(j + NBUF, b)
        return 0

    lax.fori_loop(0, D // NBUF - 1, grp_body, 0)

    for b in range(NBUF):
        wait_and_accum(D - NBUF + b, b)

    for ch in range(BW // 16):
        sl = pl.ds(ch * 16, 16)
        out_v[sl] = -acc_v[sl]
    pltpu.sync_copy(out_v, out_hbm.at[pl.ds(wid * BW, BW)])


@jax.jit
def _transe_sc(h_idx2, t_idx2, r_idx2, ent_flat, rel_flat):
    kfn = pl.kernel(
        _body,
        out_type=jax.ShapeDtypeStruct((B,), jnp.float32),
        mesh=plsc.VectorSubcoreMesh(
            core_axis_name="c", subcore_axis_name="s",
            num_cores=NC, num_subcores=NS),
        compiler_params=pltpu.CompilerParams(use_tc_tiling_on_sc=False),
        scratch_types=[
            pltpu.VMEM((NCH, IC), jnp.int32),
            pltpu.VMEM((NCH, IC), jnp.int32),
            pltpu.VMEM((NCH, IC), jnp.int32),
            pltpu.VMEM((NBUF, BW), jnp.float32),
            pltpu.VMEM((NBUF, BW), jnp.float32),
            pltpu.VMEM((NBUF, BW), jnp.float32),
            pltpu.VMEM((BW,), jnp.float32),
            pltpu.VMEM((BW,), jnp.float32),
            pltpu.SemaphoreType.DMA((NBUF,)),
        ],
    )
    return kfn(h_idx2, t_idx2, r_idx2, ent_flat, rel_flat)


def kernel(h_idx, t_idx, r_idx, entity_table, relation_table):
    ent_flat = entity_table.T.reshape(V * D)
    rel_flat = relation_table.T.reshape(R * D)
    h2 = h_idx.astype(jnp.int32).reshape(B // IC, IC)
    t2 = t_idx.astype(jnp.int32).reshape(B // IC, IC)
    r2 = r_idx.astype(jnp.int32).reshape(B // IC, IC)
    return _transe_sc(h2, t2, r2, ent_flat, rel_flat)


# speedup vs baseline: 1.0008x; 1.0008x over previous
import jax
import jax.numpy as jnp
from jax import lax
from jax.experimental import pallas as pl
from jax.experimental.pallas import tpu as pltpu
from jax.experimental.pallas import tpu_sc as plsc

B = 16384
V = 1000000
R = 1000
D = 64
NC = 2
NS = 16
NW = NC * NS
BW = B // NW
IC = 128
NCH = BW // IC
NBUF = 8


def _triples(j, b, ent_hbm, rel_hbm, hb_v, tb_v, rb_v, hd_v, td_v, rd_v,
             sems):
    out = []
    for c in range(NCH):
        sl = pl.ds(c * IC, IC)
        out.append((ent_hbm.at[j].at[hb_v.at[c]],
                    hd_v.at[b, sl], sems.at[b]))
        out.append((ent_hbm.at[j].at[tb_v.at[c]],
                    td_v.at[b, sl], sems.at[b]))
        out.append((rel_hbm.at[j].at[rb_v.at[c]],
                    rd_v.at[b, sl], sems.at[b]))
    return out


def _body(h_idx_hbm, t_idx_hbm, r_idx_hbm, ent_hbm, rel_hbm, out_hbm,
          hb_v, tb_v, rb_v, hd_v, td_v, rd_v, acc_v, out_v, sems):
    cid = lax.axis_index("c")
    sid = lax.axis_index("s")
    wid = sid * NC + cid
    row0 = wid * NCH

    pltpu.sync_copy(h_idx_hbm.at[pl.ds(row0, NCH)], hb_v)
    pltpu.sync_copy(t_idx_hbm.at[pl.ds(row0, NCH)], tb_v)
    pltpu.sync_copy(r_idx_hbm.at[pl.ds(row0, NCH)], rb_v)

    zero = jnp.zeros((16,), jnp.float32)
    for ch in range(BW // 16):
        acc_v[pl.ds(ch * 16, 16)] = zero

    def fire(j, b):
        for src, dst, sem in _triples(j, b, ent_hbm, rel_hbm, hb_v, tb_v,
                                      rb_v, hd_v, td_v, rd_v, sems):
            pltpu.async_copy(src, dst, sem)

    def wait_and_accum(j, b):
        for src, dst, sem in _triples(j, b, ent_hbm, rel_hbm, hb_v, tb_v,
                                      rb_v, hd_v, td_v, rd_v, sems):
            pltpu.make_async_copy(src, dst, sem).wait()
        for ch in range(BW // 16):
            sl = pl.ds(ch * 16, 16)
            acc_v[sl] = acc_v[sl] + jnp.abs(
                hd_v[b, sl] + rd_v[b, sl] - td_v[b, sl])

    for b in range(NBUF):
        fire(b, b)

    def grp_body(g, _):
        for b in range(NBUF):
            j = g * NBUF + b
            wait_and_accum(j, b)
            fire(j + NBUF, b)
        return 0

    lax.fori_loop(0, D // NBUF - 1, grp_body, 0)

    for b in range(NBUF):
        wait_and_accum(D - NBUF + b, b)

    for ch in range(BW // 16):
        sl = pl.ds(ch * 16, 16)
        out_v[sl] = -acc_v[sl]
    pltpu.sync_copy(out_v, out_hbm.at[pl.ds(wid * BW, BW)])


@jax.jit
def _transe_sc(h_idx2, t_idx2, r_idx2, ent_flat, rel_flat):
    kfn = pl.kernel(
        _body,
        out_type=jax.ShapeDtypeStruct((B,), jnp.float32),
        mesh=plsc.VectorSubcoreMesh(
            core_axis_name="c", subcore_axis_name="s",
            num_cores=NC, num_subcores=NS),
        compiler_params=pltpu.CompilerParams(use_tc_tiling_on_sc=False),
        scratch_types=[
            pltpu.VMEM((NCH, IC), jnp.int32),
            pltpu.VMEM((NCH, IC), jnp.int32),
            pltpu.VMEM((NCH, IC), jnp.int32),
            pltpu.VMEM((NBUF, BW), jnp.float32),
            pltpu.VMEM((NBUF, BW), jnp.float32),
            pltpu.VMEM((NBUF, BW), jnp.float32),
            pltpu.VMEM((BW,), jnp.float32),
            pltpu.VMEM((BW,), jnp.float32),
            pltpu.SemaphoreType.DMA((NBUF,)),
        ],
    )
    return kfn(h_idx2, t_idx2, r_idx2, ent_flat, rel_flat)


def kernel(h_idx, t_idx, r_idx, entity_table, relation_table):
    ent_flat = entity_table.T
    rel_flat = relation_table.T
    h2 = h_idx.astype(jnp.int32).reshape(B // IC, IC)
    t2 = t_idx.astype(jnp.int32).reshape(B // IC, IC)
    r2 = r_idx.astype(jnp.int32).reshape(B // IC, IC)
    return _transe_sc(h2, t2, r2, ent_flat, rel_flat)


# speedup vs baseline: 9.9028x; 9.8951x over previous
import functools

import jax
import jax.numpy as jnp
from jax import lax
from jax.experimental import pallas as pl
from jax.experimental.pallas import tpu as pltpu
from jax.experimental.pallas import tpu_sc as plsc

B = 16384
V = 1000000
RN = 1000
D = 64
NC = 2
NS = 16
NW = NC * NS
BW = B // NW
IC = 128
NCH = 2 * BW // IC
CS = BW * 16
NT = CS // IC
CI = 2048


def _tc_tr_body(x_ref, o_ref):
    y = jnp.swapaxes(x_ref[...], 0, 1)
    o_ref[:, 0:D] = y[0:CI // 2, :]
    o_ref[:, D:2 * D] = y[CI // 2:CI, :]


def _tc_transpose(xt, n_rows):
    grid = (n_rows + CI - 1) // CI
    return pl.pallas_call(
        _tc_tr_body,
        grid=(grid,),
        in_specs=[pl.BlockSpec((D, CI), lambda b: (0, b))],
        out_specs=pl.BlockSpec((CI // 2, 128), lambda b: (b, 0)),
        out_shape=jax.ShapeDtypeStruct((grid * CI // 2, 128), jnp.float32),
    )(xt)


def _sc_body(h_idx_hbm, t_idx_hbm, r_idx_hbm, ent_hbm, rel_hbm, perm_hbm,
             out_hbm, hi_v, ti_v, ri_v, h_v, t_v, r_v, csum_v,
             perm_v, out_v, slab, sem):
    cid = lax.axis_index("c")
    sid = lax.axis_index("s")
    wid = sid * NC + cid
    row0 = wid * NCH

    pltpu.sync_copy(h_idx_hbm.at[pl.ds(row0, NCH)], hi_v)
    pltpu.sync_copy(t_idx_hbm.at[pl.ds(row0, NCH)], ti_v)
    pltpu.sync_copy(r_idx_hbm.at[pl.ds(row0, NCH)], ri_v)
    pltpu.sync_copy(perm_hbm, perm_v)

    copies = []
    for k in range(NCH):
        copies.append(pltpu.async_copy(
            ent_hbm.at[hi_v.at[k]], h_v.at[pl.ds(k * IC, IC)], sem))
        copies.append(pltpu.async_copy(
            ent_hbm.at[ti_v.at[k]], t_v.at[pl.ds(k * IC, IC)], sem))
        copies.append(pltpu.async_copy(
            rel_hbm.at[ri_v.at[k]], r_v.at[pl.ds(k * IC, IC)], sem))
    for c in copies:
        c.wait()

    def row_body(rr, _):
        acc = None
        for k in range(2):
            for half in range(2):
                sl = pl.ds(half * 16, 16)
                d = jnp.abs(h_v[2 * rr + k, sl] + r_v[2 * rr + k, sl]
                            - t_v[2 * rr + k, sl])
                acc = d if acc is None else acc + d
        csum_v[pl.ds(rr * 16, 16)] = acc
        return 0

    lax.fori_loop(0, BW, row_body, 0)

    pltpu.sync_copy(csum_v, slab.at[sid])
    tcopies = []
    for k in range(NT):
        tcopies.append(pltpu.async_copy(
            slab.at[sid].at[perm_v.at[k]],
            csum_v.at[pl.ds(k * IC, IC)], sem))
    for c in tcopies:
        c.wait()

    def grp_body(g, _):
        acc = None
        for c in range(16):
            v = csum_v[pl.ds(c * BW + g * 16, 16)]
            acc = v if acc is None else acc + v
        out_v[pl.ds(g * 16, 16)] = -acc
        return 0

    lax.fori_loop(0, BW // 16, grp_body, 0)

    pltpu.sync_copy(out_v, out_hbm.at[pl.ds(wid * BW, BW)])


@jax.jit
def _transe(h_idx, t_idx, r_idx, entity_table, relation_table):
    ent500 = _tc_transpose(entity_table.T, V)
    rel500 = _tc_transpose(relation_table.T, RN)
    ent2 = ent500.reshape(-1, 32)
    rel2 = rel500.reshape(-1, 32)

    def _qidx(idx):
        i = idx.astype(jnp.int32)
        half = CI // 2
        q0 = (4 * half) * (i // CI) + 4 * (i % half) + 2 * ((i // half) % 2)
        return jnp.stack([q0, q0 + 1], axis=1).reshape(-1, IC)

    k = jnp.arange(CS, dtype=jnp.int32)
    perm = ((k % BW) * 16 + k // BW).reshape(NT, IC)

    kfn = pl.kernel(
        _sc_body,
        out_type=jax.ShapeDtypeStruct((B,), jnp.float32),
        mesh=plsc.VectorSubcoreMesh(
            core_axis_name="c", subcore_axis_name="s",
            num_cores=NC, num_subcores=NS),
        compiler_params=pltpu.CompilerParams(use_tc_tiling_on_sc=False),
        scratch_types=[
            pltpu.VMEM((NCH, IC), jnp.int32),
            pltpu.VMEM((NCH, IC), jnp.int32),
            pltpu.VMEM((NCH, IC), jnp.int32),
            pltpu.VMEM((2 * BW, 32), jnp.float32),
            pltpu.VMEM((2 * BW, 32), jnp.float32),
            pltpu.VMEM((2 * BW, 32), jnp.float32),
            pltpu.VMEM((CS,), jnp.float32),
            pltpu.VMEM((NT, IC), jnp.int32),
            pltpu.VMEM((BW,), jnp.float32),
            pltpu.VMEM_SHARED((NS, CS), jnp.float32),
            pltpu.SemaphoreType.DMA,
        ],
    )
    return kfn(_qidx(h_idx), _qidx(t_idx), _qidx(r_idx),
               ent2, rel2, perm)


def kernel(h_idx, t_idx, r_idx, entity_table, relation_table):
    return _transe(h_idx, t_idx, r_idx, entity_table, relation_table)


# speedup vs baseline: 15.3605x; 1.5511x over previous
import functools

import jax
import jax.numpy as jnp
from jax import lax
from jax.experimental import pallas as pl
from jax.experimental.pallas import tpu as pltpu
from jax.experimental.pallas import tpu_sc as plsc

B = 16384
V = 1000000
RN = 1000
D = 64
NC = 2
NS = 16
NW = NC * NS
BW = B // NW
IC = 128
NCH = 2 * BW // IC
CS = BW * 16
NT = CS // IC
CI = 8192


def _tc_tr_body(x_ref, o_ref):
    eye = jnp.eye(D, dtype=jnp.float32)
    y = lax.dot_general(x_ref[...], eye, (((0,), (0,)), ((), ())),
                        preferred_element_type=jnp.float32)
    o_ref[:, 0:D] = y[0:CI // 2, :]
    o_ref[:, D:2 * D] = y[CI // 2:CI, :]


def _tc_transpose(xt, n_rows):
    grid = (n_rows + CI - 1) // CI
    return pl.pallas_call(
        _tc_tr_body,
        grid=(grid,),
        in_specs=[pl.BlockSpec((D, CI), lambda b: (0, b))],
        out_specs=pl.BlockSpec((CI // 2, 128), lambda b: (b, 0)),
        out_shape=jax.ShapeDtypeStruct((grid * CI // 2, 128), jnp.float32),
    )(xt)


def _sc_body(h_idx_hbm, t_idx_hbm, r_idx_hbm, ent_hbm, rel_hbm, perm_hbm,
             out_hbm, hi_v, ti_v, ri_v, h_v, t_v, r_v, csum_v,
             perm_v, out_v, slab, sem):
    cid = lax.axis_index("c")
    sid = lax.axis_index("s")
    wid = sid * NC + cid
    row0 = wid * NCH

    pltpu.sync_copy(h_idx_hbm.at[pl.ds(row0, NCH)], hi_v)
    pltpu.sync_copy(t_idx_hbm.at[pl.ds(row0, NCH)], ti_v)
    pltpu.sync_copy(r_idx_hbm.at[pl.ds(row0, NCH)], ri_v)
    pltpu.sync_copy(perm_hbm, perm_v)

    copies = []
    for k in range(NCH):
        copies.append(pltpu.async_copy(
            ent_hbm.at[hi_v.at[k]], h_v.at[pl.ds(k * IC, IC)], sem))
        copies.append(pltpu.async_copy(
            ent_hbm.at[ti_v.at[k]], t_v.at[pl.ds(k * IC, IC)], sem))
        copies.append(pltpu.async_copy(
            rel_hbm.at[ri_v.at[k]], r_v.at[pl.ds(k * IC, IC)], sem))
    for c in copies:
        c.wait()

    def row_body(rr, _):
        acc = None
        for k in range(2):
            for half in range(2):
                sl = pl.ds(half * 16, 16)
                d = jnp.abs(h_v[2 * rr + k, sl] + r_v[2 * rr + k, sl]
                            - t_v[2 * rr + k, sl])
                acc = d if acc is None else acc + d
        csum_v[pl.ds(rr * 16, 16)] = acc
        return 0

    lax.fori_loop(0, BW, row_body, 0)

    pltpu.sync_copy(csum_v, slab.at[sid])
    tcopies = []
    for k in range(NT):
        tcopies.append(pltpu.async_copy(
            slab.at[sid].at[perm_v.at[k]],
            csum_v.at[pl.ds(k * IC, IC)], sem))
    for c in tcopies:
        c.wait()

    def grp_body(g, _):
        acc = None
        for c in range(16):
            v = csum_v[pl.ds(c * BW + g * 16, 16)]
            acc = v if acc is None else acc + v
        out_v[pl.ds(g * 16, 16)] = -acc
        return 0

    lax.fori_loop(0, BW // 16, grp_body, 0)

    pltpu.sync_copy(out_v, out_hbm.at[pl.ds(wid * BW, BW)])


@jax.jit
def _transe(h_idx, t_idx, r_idx, entity_table, relation_table):
    ent500 = _tc_transpose(entity_table.T, V)
    rel500 = _tc_transpose(relation_table.T, RN)
    ent2 = ent500.reshape(-1, 32)
    rel2 = rel500.reshape(-1, 32)

    def _qidx(idx):
        i = idx.astype(jnp.int32)
        half = CI // 2
        q0 = (4 * half) * (i // CI) + 4 * (i % half) + 2 * ((i // half) % 2)
        return jnp.stack([q0, q0 + 1], axis=1).reshape(-1, IC)

    k = jnp.arange(CS, dtype=jnp.int32)
    perm = ((k % BW) * 16 + k // BW).reshape(NT, IC)

    kfn = pl.kernel(
        _sc_body,
        out_type=jax.ShapeDtypeStruct((B,), jnp.float32),
        mesh=plsc.VectorSubcoreMesh(
            core_axis_name="c", subcore_axis_name="s",
            num_cores=NC, num_subcores=NS),
        compiler_params=pltpu.CompilerParams(use_tc_tiling_on_sc=False),
        scratch_types=[
            pltpu.VMEM((NCH, IC), jnp.int32),
            pltpu.VMEM((NCH, IC), jnp.int32),
            pltpu.VMEM((NCH, IC), jnp.int32),
            pltpu.VMEM((2 * BW, 32), jnp.float32),
            pltpu.VMEM((2 * BW, 32), jnp.float32),
            pltpu.VMEM((2 * BW, 32), jnp.float32),
            pltpu.VMEM((CS,), jnp.float32),
            pltpu.VMEM((NT, IC), jnp.int32),
            pltpu.VMEM((BW,), jnp.float32),
            pltpu.VMEM_SHARED((NS, CS), jnp.float32),
            pltpu.SemaphoreType.DMA,
        ],
    )
    return kfn(_qidx(h_idx), _qidx(t_idx), _qidx(r_idx),
               ent2, rel2, perm)


def kernel(h_idx, t_idx, r_idx, entity_table, relation_table):
    return _transe(h_idx, t_idx, r_idx, entity_table, relation_table)
